# baseline (device time: 197336 ns/iter reference)
import jax
import jax.numpy as jnp
from jax import lax
from jax.experimental import pallas as pl
from jax.experimental.pallas import tpu as pltpu

N_DEV = 32
N_ROW = 8
N_COL = 4
P = 2
NDIR = 2


def kernel(t, W):
    m, k = t.shape
    _, n = W.shape
    c1 = m // N_ROW
    c2 = c1 // N_COL
    hw = k // 2
    w = hw // P

    def col0(dir_, p):
        return dir_ * hw + p * w

    def body(t_ref, w_ref, out_ref, red1, red2,
             s1_send, s1_recv, s2_send, s2_recv,
             p1_ssem, p1_rsem, p2_ssem, p2_rsem,
             p3_ssem, p3_rsem, p4_ssem, p4_rsem):
        d = lax.axis_index("i")
        g = d // N_ROW
        r = d % N_ROW
        row_right = g * N_ROW + (r + 1) % N_ROW
        row_left = g * N_ROW + (r - 1) % N_ROW
        col_down = ((g + 1) % N_COL) * N_ROW + r
        col_up = ((g - 1) % N_COL) * N_ROW + r
        row_nbr = (row_right, row_left)
        col_nbr = (col_down, col_up)

        rho1 = (r + 1) % N_ROW
        gam1 = (g + 1) % N_COL

        def ring_rdma(src, dst, ssem, rsem, target):
            return pltpu.make_async_remote_copy(
                src_ref=src, dst_ref=dst, send_sem=ssem, recv_sem=rsem,
                device_id=(target,), device_id_type=pl.DeviceIdType.MESH,
            )

        def p1_chunk(dir_, s):
            return (r - s) % N_ROW if dir_ == 0 else (r + s + 2) % N_ROW

        def t_piece(rho, dir_, p):
            return t_ref[pl.ds(rho * c1, c1), pl.ds(col0(dir_, p), w)]

        def p1_rdma(dir_, s, p):
            return ring_rdma(
                s1_send.at[dir_, s, p], s1_recv.at[dir_, s, p],
                p1_ssem.at[dir_, s, p], p1_rsem.at[dir_, s, p],
                row_nbr[dir_])

        def p2_chunk(dir_, s):
            return (g - s) % N_COL if dir_ == 0 else (g + s + 2) % N_COL

        def red1_half(gam, dir_):
            return red1[pl.ds(gam * c2, c2), pl.ds(dir_ * hw, hw)]

        def p2_rdma(dir_, s):
            return ring_rdma(
                s2_send.at[dir_, s], s2_recv.at[dir_, s],
                p2_ssem.at[dir_, s], p2_rsem.at[dir_, s],
                col_nbr[dir_])

        def p3_rdma(dir_, h):
            gam = ((g + 1 - h) if dir_ == 0 else (g + h + 1)) % N_COL
            sl = (pl.ds(rho1 * c1 + gam * c2, c2), pl.ds(dir_ * hw, hw))
            return ring_rdma(
                out_ref.at[sl], out_ref.at[sl],
                p3_ssem.at[dir_, h], p3_rsem.at[dir_, h],
                col_nbr[dir_])

        def p4_rdma(dir_, h, p):
            rho = ((r + 1 - h) if dir_ == 0 else (r + h + 1)) % N_ROW
            sl = (pl.ds(rho * c1, c1), pl.ds(col0(dir_, p), w))
            return ring_rdma(
                out_ref.at[sl], out_ref.at[sl],
                p4_ssem.at[dir_, h, p], p4_rsem.at[dir_, h, p],
                row_nbr[dir_])

        for dir_ in range(NDIR):
            for p in range(P):
                s1_send[dir_, 0, p] = t_piece(p1_chunk(dir_, 0), dir_, p)

        barrier_sem = pltpu.get_barrier_semaphore()
        for nbr in (row_left, row_right, col_up, col_down):
            pl.semaphore_signal(
                barrier_sem, inc=1,
                device_id=(nbr,), device_id_type=pl.DeviceIdType.MESH,
            )
        pl.semaphore_wait(barrier_sem, 4)

        for dir_ in range(NDIR):
            for p in range(P):
                p1_rdma(dir_, 0, p).start()
        for s in range(N_ROW - 1):
            for p in range(P):
                for dir_ in range(NDIR):
                    rdma = p1_rdma(dir_, s, p)
                    rdma.wait_recv()
                    acc = s1_recv[dir_, s, p] + t_piece(
                        p1_chunk(dir_, s + 1), dir_, p)
                    if s < N_ROW - 2:
                        s1_send[dir_, s + 1, p] = acc
                        p1_rdma(dir_, s + 1, p).start()
                    else:
                        red1[:, pl.ds(col0(dir_, p), w)] = acc

        for dir_ in range(NDIR):
            s2_send[dir_, 0] = red1_half(p2_chunk(dir_, 0), dir_)
            p2_rdma(dir_, 0).start()
        for dir_ in range(NDIR):
            for s in range(N_ROW - 1):
                for p in range(P):
                    p1_rdma(dir_, s, p).wait_send()
        for s in range(N_COL - 1):
            for dir_ in range(NDIR):
                rdma = p2_rdma(dir_, s)
                rdma.wait_recv()
                acc = s2_recv[dir_, s] + red1_half(p2_chunk(dir_, s + 1), dir_)
                if s < N_COL - 2:
                    s2_send[dir_, s + 1] = acc
                    p2_rdma(dir_, s + 1).start()
                else:
                    red2[:, pl.ds(dir_ * hw, hw)] = acc

        own_rows = pl.ds(rho1 * c1 + gam1 * c2, c2)
        for dir_ in range(NDIR):
            out_ref[own_rows, pl.ds(dir_ * hw, hw)] = jnp.dot(
                red2[:, :], w_ref[:, pl.ds(dir_ * hw, hw)],
                preferred_element_type=jnp.float32,
            )
            p3_rdma(dir_, 0).start()

        for dir_ in range(NDIR):
            for s in range(N_COL - 1):
                p2_rdma(dir_, s).wait_send()

        for h in range(N_COL - 1):
            for dir_ in range(NDIR):
                rdma = p3_rdma(dir_, h)
                rdma.wait_recv()
                if h < N_COL - 2:
                    p3_rdma(dir_, h + 1).start()
                else:
                    for p in range(P):
                        p4_rdma(dir_, 0, p).start()

        for dir_ in range(NDIR):
            for h in range(N_COL - 1):
                p3_rdma(dir_, h).wait_send()

        for h in range(N_ROW - 1):
            for p in range(P):
                for dir_ in range(NDIR):
                    rdma = p4_rdma(dir_, h, p)
                    rdma.wait_recv()
                    if h < N_ROW - 2:
                        p4_rdma(dir_, h + 1, p).start()

        for dir_ in range(NDIR):
            for h in range(N_ROW - 1):
                for p in range(P):
                    p4_rdma(dir_, h, p).wait_send()

    return pl.pallas_call(
        body,
        out_shape=jax.ShapeDtypeStruct((m, n), jnp.float32),
        in_specs=[
            pl.BlockSpec(memory_space=pltpu.VMEM),
            pl.BlockSpec(memory_space=pltpu.VMEM),
        ],
        out_specs=pl.BlockSpec(memory_space=pltpu.VMEM),
        scratch_shapes=[
            pltpu.VMEM((c1, k), jnp.float32),
            pltpu.VMEM((c2, k), jnp.float32),
            pltpu.VMEM((NDIR, N_ROW - 1, P, c1, w), jnp.float32),
            pltpu.VMEM((NDIR, N_ROW - 1, P, c1, w), jnp.float32),
            pltpu.VMEM((NDIR, N_COL - 1, c2, hw), jnp.float32),
            pltpu.VMEM((NDIR, N_COL - 1, c2, hw), jnp.float32),
            pltpu.SemaphoreType.DMA((NDIR, N_ROW - 1, P)),
            pltpu.SemaphoreType.DMA((NDIR, N_ROW - 1, P)),
            pltpu.SemaphoreType.DMA((NDIR, N_COL - 1)),
            pltpu.SemaphoreType.DMA((NDIR, N_COL - 1)),
            pltpu.SemaphoreType.DMA((NDIR, N_COL - 1)),
            pltpu.SemaphoreType.DMA((NDIR, N_COL - 1)),
            pltpu.SemaphoreType.DMA((NDIR, N_ROW - 1, P)),
            pltpu.SemaphoreType.DMA((NDIR, N_ROW - 1, P)),
        ],
        compiler_params=pltpu.CompilerParams(collective_id=0),
    )(t, W)


# device time: 122298 ns/iter; 1.6136x vs baseline; 1.6136x over previous
import jax
import jax.numpy as jnp
from jax import lax
from jax.experimental import pallas as pl
from jax.experimental.pallas import tpu as pltpu

N_DEV = 32
N_ROW = 8
N_COL = 4
P = 2
NDIR = 2


def kernel(t, W):
    m, k = t.shape
    _, n = W.shape
    c1 = m // N_ROW
    c2 = c1 // N_COL
    hw = k // 2
    w = hw // P

    def col0(dir_, p):
        return dir_ * hw + p * w

    def body(t_ref, w_ref, out_ref, red1, red2,
             s1_send, s1_recv, s2_send, s2_recv,
             p1_ssem, p1_rsem, p2_ssem, p2_rsem,
             p3_ssem, p3_rsem, p4_ssem, p4_rsem):
        d = lax.axis_index("i")
        g = d // N_ROW
        r = d % N_ROW

        y = r // 2
        xe = r % 2
        x = jnp.where(y % 2 == 0, xe, 1 - xe)
        q = jnp.where(x == 1, y + 1, jnp.where(y == 0, 0, N_ROW - y))

        def r_of_q(qq):
            xq = jnp.where((qq >= 1) & (qq <= 4), 1, 0)
            yq = jnp.where(qq == 0, 0, jnp.where(qq <= 4, qq - 1, N_ROW - qq))
            return 2 * yq + jnp.where(yq % 2 == 0, xq, 1 - xq)

        row_right = g * N_ROW + r_of_q((q + 1) % N_ROW)
        row_left = g * N_ROW + r_of_q((q - 1) % N_ROW)
        col_down = ((g + 1) % N_COL) * N_ROW + r
        col_up = ((g - 1) % N_COL) * N_ROW + r
        row_nbr = (row_right, row_left)
        col_nbr = (col_down, col_up)

        rho1 = (q + 1) % N_ROW
        gam1 = (g + 1) % N_COL

        def ring_rdma(src, dst, ssem, rsem, target):
            return pltpu.make_async_remote_copy(
                src_ref=src, dst_ref=dst, send_sem=ssem, recv_sem=rsem,
                device_id=(target,), device_id_type=pl.DeviceIdType.MESH,
            )

        def p1_chunk(dir_, s):
            return (q - s) % N_ROW if dir_ == 0 else (q + s + 2) % N_ROW

        def t_piece(rho, dir_, p):
            return t_ref[pl.ds(rho * c1, c1), pl.ds(col0(dir_, p), w)]

        def p1_rdma(dir_, s, p):
            return ring_rdma(
                s1_send.at[dir_, s, p], s1_recv.at[dir_, s, p],
                p1_ssem.at[dir_, s, p], p1_rsem.at[dir_, s, p],
                row_nbr[dir_])

        def p2_chunk(dir_, s):
            return (g - s) % N_COL if dir_ == 0 else (g + s + 2) % N_COL

        def red1_half(gam, dir_):
            return red1[pl.ds(gam * c2, c2), pl.ds(dir_ * hw, hw)]

        def p2_rdma(dir_, s):
            return ring_rdma(
                s2_send.at[dir_, s], s2_recv.at[dir_, s],
                p2_ssem.at[dir_, s], p2_rsem.at[dir_, s],
                col_nbr[dir_])

        def p3_rdma(dir_, h):
            gam = ((g + 1 - h) if dir_ == 0 else (g + h + 1)) % N_COL
            sl = (pl.ds(rho1 * c1 + gam * c2, c2), pl.ds(dir_ * hw, hw))
            return ring_rdma(
                out_ref.at[sl], out_ref.at[sl],
                p3_ssem.at[dir_, h], p3_rsem.at[dir_, h],
                col_nbr[dir_])

        def p4_rdma(dir_, h, p):
            rho = ((q + 1 - h) if dir_ == 0 else (q + h + 1)) % N_ROW
            sl = (pl.ds(rho * c1, c1), pl.ds(col0(dir_, p), w))
            return ring_rdma(
                out_ref.at[sl], out_ref.at[sl],
                p4_ssem.at[dir_, h, p], p4_rsem.at[dir_, h, p],
                row_nbr[dir_])

        for dir_ in range(NDIR):
            for p in range(P):
                s1_send[dir_, 0, p] = t_piece(p1_chunk(dir_, 0), dir_, p)

        barrier_sem = pltpu.get_barrier_semaphore()
        for nbr in (row_left, row_right, col_up, col_down):
            pl.semaphore_signal(
                barrier_sem, inc=1,
                device_id=(nbr,), device_id_type=pl.DeviceIdType.MESH,
            )
        pl.semaphore_wait(barrier_sem, 4)

        for dir_ in range(NDIR):
            for p in range(P):
                p1_rdma(dir_, 0, p).start()
        for s in range(N_ROW - 1):
            for p in range(P):
                for dir_ in range(NDIR):
                    rdma = p1_rdma(dir_, s, p)
                    rdma.wait_recv()
                    acc = s1_recv[dir_, s, p] + t_piece(
                        p1_chunk(dir_, s + 1), dir_, p)
                    if s < N_ROW - 2:
                        s1_send[dir_, s + 1, p] = acc
                        p1_rdma(dir_, s + 1, p).start()
                    else:
                        red1[:, pl.ds(col0(dir_, p), w)] = acc

        for dir_ in range(NDIR):
            s2_send[dir_, 0] = red1_half(p2_chunk(dir_, 0), dir_)
            p2_rdma(dir_, 0).start()
        for dir_ in range(NDIR):
            for s in range(N_ROW - 1):
                for p in range(P):
                    p1_rdma(dir_, s, p).wait_send()
        for s in range(N_COL - 1):
            for dir_ in range(NDIR):
                rdma = p2_rdma(dir_, s)
                rdma.wait_recv()
                acc = s2_recv[dir_, s] + red1_half(p2_chunk(dir_, s + 1), dir_)
                if s < N_COL - 2:
                    s2_send[dir_, s + 1] = acc
                    p2_rdma(dir_, s + 1).start()
                else:
                    red2[:, pl.ds(dir_ * hw, hw)] = acc

        own_rows = pl.ds(rho1 * c1 + gam1 * c2, c2)
        for dir_ in range(NDIR):
            out_ref[own_rows, pl.ds(dir_ * hw, hw)] = jnp.dot(
                red2[:, :], w_ref[:, pl.ds(dir_ * hw, hw)],
                preferred_element_type=jnp.float32,
            )
            p3_rdma(dir_, 0).start()

        for dir_ in range(NDIR):
            for s in range(N_COL - 1):
                p2_rdma(dir_, s).wait_send()

        for h in range(N_COL - 1):
            for dir_ in range(NDIR):
                rdma = p3_rdma(dir_, h)
                rdma.wait_recv()
                if h < N_COL - 2:
                    p3_rdma(dir_, h + 1).start()
                else:
                    for p in range(P):
                        p4_rdma(dir_, 0, p).start()

        for dir_ in range(NDIR):
            for h in range(N_COL - 1):
                p3_rdma(dir_, h).wait_send()

        for h in range(N_ROW - 1):
            for p in range(P):
                for dir_ in range(NDIR):
                    rdma = p4_rdma(dir_, h, p)
                    rdma.wait_recv()
                    if h < N_ROW - 2:
                        p4_rdma(dir_, h + 1, p).start()

        for dir_ in range(NDIR):
            for h in range(N_ROW - 1):
                for p in range(P):
                    p4_rdma(dir_, h, p).wait_send()

    return pl.pallas_call(
        body,
        out_shape=jax.ShapeDtypeStruct((m, n), jnp.float32),
        in_specs=[
            pl.BlockSpec(memory_space=pltpu.VMEM),
            pl.BlockSpec(memory_space=pltpu.VMEM),
        ],
        out_specs=pl.BlockSpec(memory_space=pltpu.VMEM),
        scratch_shapes=[
            pltpu.VMEM((c1, k), jnp.float32),
            pltpu.VMEM((c2, k), jnp.float32),
            pltpu.VMEM((NDIR, N_ROW - 1, P, c1, w), jnp.float32),
            pltpu.VMEM((NDIR, N_ROW - 1, P, c1, w), jnp.float32),
            pltpu.VMEM((NDIR, N_COL - 1, c2, hw), jnp.float32),
            pltpu.VMEM((NDIR, N_COL - 1, c2, hw), jnp.float32),
            pltpu.SemaphoreType.DMA((NDIR, N_ROW - 1, P)),
            pltpu.SemaphoreType.DMA((NDIR, N_ROW - 1, P)),
            pltpu.SemaphoreType.DMA((NDIR, N_COL - 1)),
            pltpu.SemaphoreType.DMA((NDIR, N_COL - 1)),
            pltpu.SemaphoreType.DMA((NDIR, N_COL - 1)),
            pltpu.SemaphoreType.DMA((NDIR, N_COL - 1)),
            pltpu.SemaphoreType.DMA((NDIR, N_ROW - 1, P)),
            pltpu.SemaphoreType.DMA((NDIR, N_ROW - 1, P)),
        ],
        compiler_params=pltpu.CompilerParams(collective_id=0),
    )(t, W)


# device time: 119632 ns/iter; 1.6495x vs baseline; 1.0223x over previous
import jax
import jax.numpy as jnp
from jax import lax
from jax.experimental import pallas as pl
from jax.experimental.pallas import tpu as pltpu

N_DEV = 32
N_ROW = 8
N_COL = 4
P = 2
NDIR = 2


def kernel(t, W):
    m, k = t.shape
    _, n = W.shape
    c1 = m // N_ROW
    c2 = c1 // N_COL
    hw = k // 2
    w = hw // P

    def col0(dir_, p):
        return dir_ * hw + p * w

    def body(t_ref, w_ref, out_ref, red1, red2,
             s1_send, s1_recv, z2_recv,
             p1_ssem, p1_rsem, p2_ssem, p2_rsem,
             p3_ssem, p3_rsem, p4_ssem, p4_rsem):
        d = lax.axis_index("i")
        g = d // N_ROW
        r = d % N_ROW

        y = r // 2
        xe = r % 2
        x = jnp.where(y % 2 == 0, xe, 1 - xe)
        q = jnp.where(x == 1, y + 1, jnp.where(y == 0, 0, N_ROW - y))

        def r_of_q(qq):
            xq = jnp.where((qq >= 1) & (qq <= 4), 1, 0)
            yq = jnp.where(qq == 0, 0, jnp.where(qq <= 4, qq - 1, N_ROW - qq))
            return 2 * yq + jnp.where(yq % 2 == 0, xq, 1 - xq)

        row_right = g * N_ROW + r_of_q((q + 1) % N_ROW)
        row_left = g * N_ROW + r_of_q((q - 1) % N_ROW)
        row_nbr = (row_right, row_left)

        def z_peer(a):
            return ((g + a) % N_COL) * N_ROW + r

        rho1 = (q + 1) % N_ROW
        gam1 = (g + 1) % N_COL

        def ring_rdma(src, dst, ssem, rsem, target):
            return pltpu.make_async_remote_copy(
                src_ref=src, dst_ref=dst, send_sem=ssem, recv_sem=rsem,
                device_id=(target,), device_id_type=pl.DeviceIdType.MESH,
            )

        def p1_chunk(dir_, s):
            return (q - s) % N_ROW if dir_ == 0 else (q + s + 2) % N_ROW

        def t_piece(rho, dir_, p):
            return t_ref[pl.ds(rho * c1, c1), pl.ds(col0(dir_, p), w)]

        def p1_rdma(dir_, s, p):
            return ring_rdma(
                s1_send.at[dir_, s, p], s1_recv.at[dir_, s, p],
                p1_ssem.at[dir_, s, p], p1_rsem.at[dir_, s, p],
                row_nbr[dir_])

        def p2_send_rdma(dir_, a):
            gam = (g + a + 1) % N_COL
            return ring_rdma(
                red1.at[pl.ds(gam * c2, c2), pl.ds(dir_ * hw, hw)],
                z2_recv.at[dir_, N_COL - a],
                p2_ssem.at[dir_, a], p2_rsem.at[dir_, N_COL - a],
                z_peer(a))

        def p2_recv_rdma(dir_, jj):
            return ring_rdma(
                z2_recv.at[dir_, jj], z2_recv.at[dir_, jj],
                p2_ssem.at[dir_, jj], p2_rsem.at[dir_, jj],
                z_peer(jj))

        own_rows = pl.ds(rho1 * c1 + gam1 * c2, c2)

        def p3_send_rdma(dir_, a):
            sl = (own_rows, pl.ds(dir_ * hw, hw))
            return ring_rdma(
                out_ref.at[sl], out_ref.at[sl],
                p3_ssem.at[dir_, a], p3_rsem.at[dir_, N_COL - a],
                z_peer(a))

        def p3_recv_rdma(dir_, jj):
            gam = (g + jj + 1) % N_COL
            sl = (pl.ds(rho1 * c1 + gam * c2, c2), pl.ds(dir_ * hw, hw))
            return ring_rdma(
                out_ref.at[sl], out_ref.at[sl],
                p3_ssem.at[dir_, jj], p3_rsem.at[dir_, jj],
                z_peer(jj))

        def p4_rdma(dir_, h, p):
            rho = ((q + 1 - h) if dir_ == 0 else (q + h + 1)) % N_ROW
            sl = (pl.ds(rho * c1, c1), pl.ds(col0(dir_, p), w))
            return ring_rdma(
                out_ref.at[sl], out_ref.at[sl],
                p4_ssem.at[dir_, h, p], p4_rsem.at[dir_, h, p],
                row_nbr[dir_])

        for dir_ in range(NDIR):
            for p in range(P):
                s1_send[dir_, 0, p] = t_piece(p1_chunk(dir_, 0), dir_, p)

        barrier_sem = pltpu.get_barrier_semaphore()
        for nbr in (row_left, row_right, z_peer(1), z_peer(2), z_peer(3)):
            pl.semaphore_signal(
                barrier_sem, inc=1,
                device_id=(nbr,), device_id_type=pl.DeviceIdType.MESH,
            )
        pl.semaphore_wait(barrier_sem, 5)

        for dir_ in range(NDIR):
            for p in range(P):
                p1_rdma(dir_, 0, p).start()
        for s in range(N_ROW - 1):
            for p in range(P):
                for dir_ in range(NDIR):
                    rdma = p1_rdma(dir_, s, p)
                    rdma.wait_recv()
                    acc = s1_recv[dir_, s, p] + t_piece(
                        p1_chunk(dir_, s + 1), dir_, p)
                    if s < N_ROW - 2:
                        s1_send[dir_, s + 1, p] = acc
                        p1_rdma(dir_, s + 1, p).start()
                    else:
                        red1[:, pl.ds(col0(dir_, p), w)] = acc

        for a in range(1, N_COL):
            for dir_ in range(NDIR):
                p2_send_rdma(dir_, a).start()
        for dir_ in range(NDIR):
            for s in range(N_ROW - 1):
                for p in range(P):
                    p1_rdma(dir_, s, p).wait_send()
        for dir_ in range(NDIR):
            for jj in range(1, N_COL):
                p2_recv_rdma(dir_, jj).wait_recv()
            red2[:, pl.ds(dir_ * hw, hw)] = (
                (red1[pl.ds(gam1 * c2, c2), pl.ds(dir_ * hw, hw)]
                 + z2_recv[dir_, 1])
                + (z2_recv[dir_, 2] + z2_recv[dir_, 3])
            )

        for dir_ in range(NDIR):
            out_ref[own_rows, pl.ds(dir_ * hw, hw)] = jnp.dot(
                red2[:, :], w_ref[:, pl.ds(dir_ * hw, hw)],
                preferred_element_type=jnp.float32,
            )
            for a in range(1, N_COL):
                p3_send_rdma(dir_, a).start()

        for a in range(1, N_COL):
            for dir_ in range(NDIR):
                p2_send_rdma(dir_, a).wait_send()

        for dir_ in range(NDIR):
            for jj in range(1, N_COL):
                p3_recv_rdma(dir_, jj).wait_recv()
            for p in range(P):
                p4_rdma(dir_, 0, p).start()

        for a in range(1, N_COL):
            for dir_ in range(NDIR):
                p3_send_rdma(dir_, a).wait_send()

        for h in range(N_ROW - 1):
            for p in range(P):
                for dir_ in range(NDIR):
                    rdma = p4_rdma(dir_, h, p)
                    rdma.wait_recv()
                    if h < N_ROW - 2:
                        p4_rdma(dir_, h + 1, p).start()

        for dir_ in range(NDIR):
            for h in range(N_ROW - 1):
                for p in range(P):
                    p4_rdma(dir_, h, p).wait_send()

    return pl.pallas_call(
        body,
        out_shape=jax.ShapeDtypeStruct((m, n), jnp.float32),
        in_specs=[
            pl.BlockSpec(memory_space=pltpu.VMEM),
            pl.BlockSpec(memory_space=pltpu.VMEM),
        ],
        out_specs=pl.BlockSpec(memory_space=pltpu.VMEM),
        scratch_shapes=[
            pltpu.VMEM((c1, k), jnp.float32),
            pltpu.VMEM((c2, k), jnp.float32),
            pltpu.VMEM((NDIR, N_ROW - 1, P, c1, w), jnp.float32),
            pltpu.VMEM((NDIR, N_ROW - 1, P, c1, w), jnp.float32),
            pltpu.VMEM((NDIR, N_COL, c2, hw), jnp.float32),
            pltpu.SemaphoreType.DMA((NDIR, N_ROW - 1, P)),
            pltpu.SemaphoreType.DMA((NDIR, N_ROW - 1, P)),
            pltpu.SemaphoreType.DMA((NDIR, N_COL)),
            pltpu.SemaphoreType.DMA((NDIR, N_COL)),
            pltpu.SemaphoreType.DMA((NDIR, N_COL)),
            pltpu.SemaphoreType.DMA((NDIR, N_COL)),
            pltpu.SemaphoreType.DMA((NDIR, N_ROW - 1, P)),
            pltpu.SemaphoreType.DMA((NDIR, N_ROW - 1, P)),
        ],
        compiler_params=pltpu.CompilerParams(collective_id=0),
    )(t, W)
